# Initial kernel scaffold; baseline (speedup 1.0000x reference)
import jax
import jax.numpy as jnp
from jax.experimental import pallas as pl


def kernel(logits, edges_pc, weight):
    raise NotImplementedError("write your pallas kernel here")


# speedup vs baseline: 1.7100x; 1.7100x over previous
import functools

import jax
import jax.numpy as jnp
from jax import lax
from jax.experimental import pallas as pl
from jax.experimental.pallas import tpu as pltpu
from jax.experimental.pallas import tpu_sc as plsc

_MARGIN = 0.05

_ROWS = 16384
_COLS = 1000
_E = 2000

_NC = 2
_NS = 16
_L = 16
_NW = _NC * _NS
_RPW = _ROWS // _NW
_R = 32
_NCHUNK = _RPW // _R
_NG = _E // _L
_UNROLL = 4

_mesh = plsc.VectorSubcoreMesh(core_axis_name="c", subcore_axis_name="s")


@functools.partial(
    pl.kernel,
    mesh=_mesh,
    out_type=jax.ShapeDtypeStruct((_NW, _L), jnp.float32),
    compiler_params=pltpu.CompilerParams(needs_layout_passes=False),
    scratch_types=[
        pltpu.VMEM((_E,), jnp.int32),
        pltpu.VMEM((_E,), jnp.int32),
        pltpu.VMEM((_E,), jnp.float32),
        pltpu.VMEM((_R * _COLS,), jnp.float32),
        pltpu.VMEM((_R * _COLS,), jnp.float32),
        pltpu.VMEM((_L,), jnp.float32),
        pltpu.SemaphoreType.DMA,
        pltpu.SemaphoreType.DMA,
    ],
)
def _hcl_sc(logits_hbm, cidx_hbm, pidx_hbm, w_hbm, out_hbm,
            cidx_v, pidx_v, w_v, buf0, buf1, out_v, sem0, sem1):
    wid = lax.axis_index("s") * _NC + lax.axis_index("c")
    base_word = wid * (_RPW * _COLS)

    pltpu.sync_copy(cidx_hbm, cidx_v)
    pltpu.sync_copy(pidx_hbm, pidx_v)
    pltpu.sync_copy(w_hbm, w_v)

    bufs = (buf0, buf1)
    sems = (sem0, sem1)
    copies = [None, None]
    copies[0] = pltpu.async_copy(
        logits_hbm.at[pl.ds(base_word, _R * _COLS)], buf0, sem0)

    zero = jnp.zeros((_L,), jnp.float32)
    neg_m = jnp.full((_L,), -_MARGIN, jnp.float32)
    accs = (zero, zero, zero, zero)

    for k in range(_NCHUNK):
        if k + 1 < _NCHUNK:
            nxt = (k + 1) % 2
            copies[nxt] = pltpu.async_copy(
                logits_hbm.at[pl.ds(base_word + (k + 1) * _R * _COLS,
                                    _R * _COLS)],
                bufs[nxt], sems[nxt])
        copies[k % 2].wait()
        buf = bufs[k % 2]

        def group_body(g, accs4, buf=buf):
            cvec = cidx_v[pl.ds(g * _L, _L)]
            pvec = pidx_v[pl.ds(g * _L, _L)]
            wvec = w_v[pl.ds(g * _L, _L)]

            def row_body(j, st):
                a0, a1, a2, a3, ca, pa = st
                outs = []
                for u in range(_UNROLL):
                    cu = plsc.load_gather(buf, [ca + u * _COLS])
                    pu = plsc.load_gather(buf, [pa + u * _COLS])
                    outs.append(jnp.maximum(cu - pu, neg_m) * wvec)
                return (a0 + outs[0], a1 + outs[1], a2 + outs[2],
                        a3 + outs[3],
                        ca + _UNROLL * _COLS, pa + _UNROLL * _COLS)

            st = lax.fori_loop(
                0, _R // _UNROLL, row_body, accs4 + (cvec, pvec))
            return st[:4]

        accs = lax.fori_loop(0, _NG, group_body, accs)

    def wsum_body(g, s):
        return s + w_v[pl.ds(g * _L, _L)]
    wsum = lax.fori_loop(0, _NG, wsum_body, zero)

    total = (accs[0] + accs[1]) + (accs[2] + accs[3])
    total = total + (_MARGIN * _RPW) * wsum
    out_v[...] = total
    pltpu.sync_copy(out_v, out_hbm.at[wid])


def kernel(logits, edges_pc, weight):
    cidx = edges_pc[1].astype(jnp.int32)
    pidx = edges_pc[0].astype(jnp.int32)
    partials = _hcl_sc(logits.reshape(-1), cidx, pidx,
                       weight.astype(jnp.float32))
    return jnp.sum(partials) / (_ROWS * _E)


# speedup vs baseline: 2.7532x; 1.6100x over previous
import functools

import jax
import jax.numpy as jnp
from jax import lax
from jax.experimental import pallas as pl
from jax.experimental.pallas import tpu as pltpu
from jax.experimental.pallas import tpu_sc as plsc

_MARGIN = 0.05

_ROWS = 16384
_COLS = 1000
_E = 2000

_TC_ROWS = 8192
_SC_ROWS = _ROWS - _TC_ROWS

_NC = 2
_NS = 16
_L = 16
_NW = _NC * _NS
_RPW = _SC_ROWS // _NW
_R = 32
_NCHUNK = _RPW // _R
_NG = _E // _L

_BM = 512
_NBLK = _TC_ROWS // _BM

_mesh = plsc.VectorSubcoreMesh(core_axis_name="c", subcore_axis_name="s")


@functools.partial(
    pl.kernel,
    mesh=_mesh,
    out_type=jax.ShapeDtypeStruct((_NW, _L), jnp.float32),
    compiler_params=pltpu.CompilerParams(needs_layout_passes=False),
    scratch_types=[
        pltpu.VMEM((_E,), jnp.int32),
        pltpu.VMEM((_E,), jnp.int32),
        pltpu.VMEM((_E,), jnp.float32),
        pltpu.VMEM((_R, _COLS), jnp.float32),
        pltpu.VMEM((_R, _COLS), jnp.float32),
        pltpu.VMEM((_L,), jnp.float32),
        pltpu.SemaphoreType.DMA,
        pltpu.SemaphoreType.DMA,
    ],
)
def _hcl_sc(logits_hbm, cidx_hbm, pidx_hbm, w_hbm, out_hbm,
            cidx_v, pidx_v, w_v, buf0, buf1, out_v, sem0, sem1):
    wid = lax.axis_index("s") * _NC + lax.axis_index("c")
    row_base = _TC_ROWS + wid * _RPW

    pltpu.sync_copy(cidx_hbm, cidx_v)
    pltpu.sync_copy(pidx_hbm, pidx_v)
    pltpu.sync_copy(w_hbm, w_v)

    bufs = (buf0, buf1)
    sems = (sem0, sem1)
    copies = [None, None]
    copies[0] = pltpu.async_copy(
        logits_hbm.at[pl.ds(row_base, _R), :], buf0, sem0)

    zero = jnp.zeros((_L,), jnp.float32)
    neg_m = jnp.full((_L,), -_MARGIN, jnp.float32)
    accs = (zero, zero, zero, zero)

    for k in range(_NCHUNK):
        if k + 1 < _NCHUNK:
            nxt = (k + 1) % 2
            copies[nxt] = pltpu.async_copy(
                logits_hbm.at[pl.ds(row_base + (k + 1) * _R, _R), :],
                bufs[nxt], sems[nxt])
        copies[k % 2].wait()
        buf = bufs[k % 2]

        for half in range(_R // 16):

            def group_body(g, accs4, buf=buf, r0=half * 16):
                cvec = cidx_v[pl.ds(g * _L, _L)]
                pvec = pidx_v[pl.ds(g * _L, _L)]
                wvec = w_v[pl.ds(g * _L, _L)]
                a0, a1, a2, a3 = accs4
                for u in range(r0, r0 + 16):
                    ru = jnp.full((_L,), u, jnp.int32)
                    cu = plsc.load_gather(buf, [ru, cvec])
                    pu = plsc.load_gather(buf, [ru, pvec])
                    t = jnp.maximum(cu - pu, neg_m) * wvec
                    a0, a1, a2, a3 = a1, a2, a3, a0 + t
                return (a0, a1, a2, a3)

            accs = lax.fori_loop(0, _NG, group_body, accs)

    def wsum_body(g, s):
        return s + w_v[pl.ds(g * _L, _L)]
    wsum = lax.fori_loop(0, _NG, wsum_body, zero)

    total = (accs[0] + accs[1]) + (accs[2] + accs[3])
    total = total + (_MARGIN * _RPW) * wsum
    out_v[...] = total
    pltpu.sync_copy(out_v, out_hbm.at[wid])


def _hcl_tc_body(logits_ref, cidx_ref, pidx_ref, w_ref, out_ref,
                 g_ref, acc_ref):
    i = pl.program_id(0)

    @pl.when(i == 0)
    def _build_g():
        iota = lax.broadcasted_iota(jnp.int32, (_COLS, _E), 0)
        gm = (iota == cidx_ref[...]).astype(jnp.bfloat16)
        g_ref[...] = gm - (iota == pidx_ref[...]).astype(jnp.bfloat16)
        acc_ref[0, 0] = 0.0

    x = logits_ref[...]
    hi = x.astype(jnp.bfloat16)
    lo = (x - hi.astype(jnp.float32)).astype(jnp.bfloat16)
    gm = g_ref[...]
    d = (jnp.dot(hi, gm, preferred_element_type=jnp.float32)
         + jnp.dot(lo, gm, preferred_element_type=jnp.float32))
    t = jnp.maximum(d + _MARGIN, 0.0) * w_ref[...]
    acc_ref[0, 0] += jnp.sum(t)

    @pl.when(i == _NBLK - 1)
    def _emit():
        out_ref[0, 0] = acc_ref[0, 0]


_hcl_tc = pl.pallas_call(
    _hcl_tc_body,
    grid=(_NBLK,),
    in_specs=[
        pl.BlockSpec((_BM, _COLS), lambda i: (i, 0)),
        pl.BlockSpec((1, _E), lambda i: (0, 0)),
        pl.BlockSpec((1, _E), lambda i: (0, 0)),
        pl.BlockSpec((1, _E), lambda i: (0, 0)),
    ],
    out_specs=pl.BlockSpec(memory_space=pltpu.SMEM),
    out_shape=jax.ShapeDtypeStruct((1, 1), jnp.float32),
    scratch_shapes=[
        pltpu.VMEM((_COLS, _E), jnp.bfloat16),
        pltpu.SMEM((1, 1), jnp.float32),
    ],
)


def kernel(logits, edges_pc, weight):
    cidx = edges_pc[1].astype(jnp.int32)
    pidx = edges_pc[0].astype(jnp.int32)
    w32 = weight.astype(jnp.float32)
    sc_partials = _hcl_sc(logits, cidx, pidx, w32)
    tc_partial = _hcl_tc(logits, cidx[None, :], pidx[None, :], w32[None, :])
    total = jnp.sum(sc_partials) + tc_partial[0, 0]
    return total / (_ROWS * _E)
